# Initial kernel scaffold; baseline (speedup 1.0000x reference)
import functools

import jax
import jax.numpy as jnp
from jax.experimental import pallas as pl


def _route_kernel(x_ref, wg_ref, idx_ref, *, T, E, CAP):
    x = x_ref[...]
    logits = jnp.dot(x, wg_ref[...], preferred_element_type=jnp.float32)
    m = jnp.max(logits, axis=1, keepdims=True)
    iota_e = jax.lax.broadcasted_iota(jnp.int32, (T, E), 1)
    a = jnp.min(jnp.where(logits == m, iota_e, E), axis=1, keepdims=True)
    oh = (iota_e == a).astype(jnp.float32)
    c = oh
    s = 1
    while s < T:
        shifted = jnp.concatenate(
            [jnp.zeros((s, E), jnp.float32), c[: T - s]], axis=0)
        c = c + shifted
        s *= 2
    slot = jnp.sum(c * oh, axis=1, keepdims=True) - 1.0
    kept = slot < CAP
    j = a * CAP + slot.astype(jnp.int32)
    CH = 256
    tf = jnp.zeros((1, E * CAP), jnp.float32)
    iota_j = jax.lax.broadcasted_iota(jnp.int32, (CH, E * CAP), 1)
    for k in range(T // CH):
        jc = jax.lax.slice_in_dim(j, k * CH, (k + 1) * CH, axis=0)
        kc = jax.lax.slice_in_dim(kept, k * CH, (k + 1) * CH, axis=0)
        tvals = jax.lax.broadcasted_iota(jnp.float32, (CH, 1), 0) + (k * CH + 1)
        onehot = (iota_j == jc) & kc
        tf = tf + jnp.sum(jnp.where(onehot, tvals, 0.0), axis=0, keepdims=True)
    idx_ref[...] = jnp.where(tf > 0.5, tf.astype(jnp.int32) - 1, T)


def _expert_kernel(idx_ref, x_ref, w1_ref, b1_ref, w2_ref, b2_ref, out_ref, *, T):
    e = pl.program_id(0)
    idxv = idx_ref[0]
    iota_t = jax.lax.broadcasted_iota(jnp.int32, (T, idxv.shape[1]), 0)
    p = (iota_t == idxv).astype(jnp.float32)
    xs = jax.lax.dot_general(
        p, x_ref[...], (((0,), (0,)), ((), ())),
        preferred_element_type=jnp.float32)
    h = jnp.maximum(
        jnp.dot(xs, w1_ref[0], preferred_element_type=jnp.float32) + b1_ref[0],
        0.0)
    ys = jnp.dot(h, w2_ref[0], preferred_element_type=jnp.float32) + b2_ref[0]
    contrib = jnp.dot(p, ys, preferred_element_type=jnp.float32)

    @pl.when(e == 0)
    def _():
        out_ref[...] = contrib

    @pl.when(e > 0)
    def _():
        out_ref[...] += contrib


def kernel(x, Wg, W1, b1, W2, b2):
    T, D = x.shape
    E = Wg.shape[1]
    H = W1.shape[2]
    CAP = max(1, (2 * T) // E)

    idx_flat = pl.pallas_call(
        functools.partial(_route_kernel, T=T, E=E, CAP=CAP),
        out_shape=jax.ShapeDtypeStruct((1, E * CAP), jnp.int32),
    )(x, Wg)
    idx3 = idx_flat.reshape(E, 1, CAP)
    b1r = b1.reshape(E, 1, H)
    b2r = b2.reshape(E, 1, D)

    out = pl.pallas_call(
        functools.partial(_expert_kernel, T=T),
        grid=(E,),
        in_specs=[
            pl.BlockSpec((1, 1, CAP), lambda e: (e, 0, 0)),
            pl.BlockSpec((T, D), lambda e: (0, 0)),
            pl.BlockSpec((1, D, H), lambda e: (e, 0, 0)),
            pl.BlockSpec((1, 1, H), lambda e: (e, 0, 0)),
            pl.BlockSpec((1, H, D), lambda e: (e, 0, 0)),
            pl.BlockSpec((1, 1, D), lambda e: (e, 0, 0)),
        ],
        out_specs=pl.BlockSpec((T, D), lambda e: (0, 0)),
        out_shape=jax.ShapeDtypeStruct((T, D), jnp.float32),
    )(idx3, x, W1, b1r, W2, b2r)
    return out


# speedup vs baseline: 7.7086x; 7.7086x over previous
import functools

import jax
import jax.numpy as jnp
from jax.experimental import pallas as pl


def _route_kernel(x_ref, wg_ref, idx_ref, *, T, E, CAP):
    x = x_ref[...]
    logits = jnp.dot(x, wg_ref[...], preferred_element_type=jnp.float32)
    m = jnp.max(logits, axis=1, keepdims=True)
    iota_e = jax.lax.broadcasted_iota(jnp.int32, (T, E), 1)
    a = jnp.min(jnp.where(logits == m, iota_e, E), axis=1, keepdims=True)
    oh = (iota_e == a).astype(jnp.float32)
    c = oh
    s = 1
    while s < T:
        shifted = jnp.concatenate(
            [jnp.zeros((s, E), jnp.float32), c[: T - s]], axis=0)
        c = c + shifted
        s *= 2
    slot = jnp.sum(c * oh, axis=1, keepdims=True) - 1.0
    kept = slot < CAP
    j = a * CAP + slot.astype(jnp.int32)
    CH = 256
    tf = jnp.zeros((1, E * CAP), jnp.float32)
    iota_j = jax.lax.broadcasted_iota(jnp.int32, (CH, E * CAP), 1)
    for k in range(T // CH):
        jc = jax.lax.slice_in_dim(j, k * CH, (k + 1) * CH, axis=0)
        kc = jax.lax.slice_in_dim(kept, k * CH, (k + 1) * CH, axis=0)
        tvals = (jax.lax.broadcasted_iota(jnp.int32, (CH, 1), 0)
                 + (k * CH + 1)).astype(jnp.float32)
        onehot = (iota_j == jc) & kc
        tf = tf + jnp.sum(jnp.where(onehot, tvals, 0.0), axis=0, keepdims=True)
    idx_ref[...] = jnp.where(tf > 0.5, tf.astype(jnp.int32) - 1, T)


def _expert_kernel(idx_ref, x_ref, w1_ref, b1_ref, w2_ref, b2_ref, out_ref, *, T):
    e = pl.program_id(0)
    idxv = idx_ref[0]
    iota_t = jax.lax.broadcasted_iota(jnp.int32, (T, idxv.shape[1]), 0)
    p = (iota_t == idxv).astype(jnp.float32)
    xs = jax.lax.dot_general(
        p, x_ref[...], (((0,), (0,)), ((), ())),
        preferred_element_type=jnp.float32)
    h = jnp.maximum(
        jnp.dot(xs, w1_ref[0], preferred_element_type=jnp.float32) + b1_ref[0],
        0.0)
    ys = jnp.dot(h, w2_ref[0], preferred_element_type=jnp.float32) + b2_ref[0]
    contrib = jnp.dot(p, ys, preferred_element_type=jnp.float32)

    @pl.when(e == 0)
    def _():
        out_ref[...] = contrib

    @pl.when(e > 0)
    def _():
        out_ref[...] += contrib


def kernel(x, Wg, W1, b1, W2, b2):
    T, D = x.shape
    E = Wg.shape[1]
    H = W1.shape[2]
    CAP = max(1, (2 * T) // E)

    idx_flat = pl.pallas_call(
        functools.partial(_route_kernel, T=T, E=E, CAP=CAP),
        out_shape=jax.ShapeDtypeStruct((1, E * CAP), jnp.int32),
    )(x, Wg)
    idx3 = idx_flat.reshape(E, 1, CAP)
    b1r = b1.reshape(E, 1, H)
    b2r = b2.reshape(E, 1, D)

    out = pl.pallas_call(
        functools.partial(_expert_kernel, T=T),
        grid=(E,),
        in_specs=[
            pl.BlockSpec((1, 1, CAP), lambda e: (e, 0, 0)),
            pl.BlockSpec((T, D), lambda e: (0, 0)),
            pl.BlockSpec((1, D, H), lambda e: (e, 0, 0)),
            pl.BlockSpec((1, 1, H), lambda e: (e, 0, 0)),
            pl.BlockSpec((1, H, D), lambda e: (e, 0, 0)),
            pl.BlockSpec((1, 1, D), lambda e: (e, 0, 0)),
        ],
        out_specs=pl.BlockSpec((T, D), lambda e: (0, 0)),
        out_shape=jax.ShapeDtypeStruct((T, D), jnp.float32),
    )(idx3, x, W1, b1r, W2, b2r)
    return out


# speedup vs baseline: 8.0714x; 1.0471x over previous
import functools

import jax
import jax.numpy as jnp
from jax.experimental import pallas as pl
from jax.experimental.pallas import tpu as pltpu


def _moe_kernel(x_ref, wg_ref, w1_ref, b1_ref, w2_ref, b2_ref, out_ref,
                idx_scr, *, T, E, CAP):
    e = pl.program_id(0)

    @pl.when(e == 0)
    def _route():
        x = x_ref[...]
        logits = jnp.dot(x, wg_ref[...], preferred_element_type=jnp.float32)
        m = jnp.max(logits, axis=1, keepdims=True)
        iota_e = jax.lax.broadcasted_iota(jnp.int32, (T, E), 1)
        a = jnp.min(jnp.where(logits == m, iota_e, E), axis=1, keepdims=True)
        oh = (iota_e == a).astype(jnp.float32)
        c = oh
        s = 1
        while s < T:
            shifted = jnp.concatenate(
                [jnp.zeros((s, E), jnp.float32), c[: T - s]], axis=0)
            c = c + shifted
            s *= 2
        slot = jnp.sum(c * oh, axis=1, keepdims=True) - 1.0
        kept = slot < CAP
        j = a * CAP + slot.astype(jnp.int32)
        CH = 256
        tf = jnp.zeros((1, E * CAP), jnp.float32)
        iota_j = jax.lax.broadcasted_iota(jnp.int32, (CH, E * CAP), 1)
        for k in range(T // CH):
            jc = jax.lax.slice_in_dim(j, k * CH, (k + 1) * CH, axis=0)
            kc = jax.lax.slice_in_dim(kept, k * CH, (k + 1) * CH, axis=0)
            tvals = (jax.lax.broadcasted_iota(jnp.int32, (CH, 1), 0)
                     + (k * CH + 1)).astype(jnp.float32)
            onehot = (iota_j == jc) & kc
            tf = tf + jnp.sum(jnp.where(onehot, tvals, 0.0), axis=0,
                              keepdims=True)
        tf_i = jnp.where(tf > 0.5, tf.astype(jnp.int32) - 1, T)
        for ee in range(E):
            idx_scr[pl.ds(ee, 1), :] = tf_i[:, ee * CAP:(ee + 1) * CAP]

    idxv = idx_scr[pl.ds(e, 1), :]
    iota_t = jax.lax.broadcasted_iota(jnp.int32, (T, CAP), 0)
    p = (iota_t == idxv).astype(jnp.float32)
    xs = jax.lax.dot_general(
        p, x_ref[...], (((0,), (0,)), ((), ())),
        preferred_element_type=jnp.float32)
    h = jnp.maximum(
        jnp.dot(xs, w1_ref[0], preferred_element_type=jnp.float32) + b1_ref[0],
        0.0)
    ys = jnp.dot(h, w2_ref[0], preferred_element_type=jnp.float32) + b2_ref[0]
    contrib = jnp.dot(p, ys, preferred_element_type=jnp.float32)

    @pl.when(e == 0)
    def _init():
        out_ref[...] = contrib

    @pl.when(e > 0)
    def _acc():
        out_ref[...] += contrib


def kernel(x, Wg, W1, b1, W2, b2):
    T, D = x.shape
    E = Wg.shape[1]
    H = W1.shape[2]
    CAP = max(1, (2 * T) // E)
    b1r = b1.reshape(E, 1, H)
    b2r = b2.reshape(E, 1, D)

    return pl.pallas_call(
        functools.partial(_moe_kernel, T=T, E=E, CAP=CAP),
        grid=(E,),
        in_specs=[
            pl.BlockSpec((T, D), lambda e: (0, 0)),
            pl.BlockSpec((D, E), lambda e: (0, 0)),
            pl.BlockSpec((1, D, H), lambda e: (e, 0, 0)),
            pl.BlockSpec((1, 1, H), lambda e: (e, 0, 0)),
            pl.BlockSpec((1, H, D), lambda e: (e, 0, 0)),
            pl.BlockSpec((1, 1, D), lambda e: (e, 0, 0)),
        ],
        out_specs=pl.BlockSpec((T, D), lambda e: (0, 0)),
        out_shape=jax.ShapeDtypeStruct((T, D), jnp.float32),
        scratch_shapes=[pltpu.VMEM((E, CAP), jnp.int32)],
    )(x, Wg, W1, b1r, W2, b2r)
